# baseline (speedup 1.0000x reference)
import functools

import jax
import jax.numpy as jnp
from jax import lax
from jax.experimental import pallas as pl
from jax.experimental.pallas import tpu as pltpu
from jax.experimental.pallas import tpu_sc as plsc

N = 10000
N_PAD = 10240
F_IN = 128
F_H = 64
F_OUT = 32
NC = 2
NS = 16
NW = NC * NS
CHUNK = 125
RPT = N_PAD // NS
DEG_W = 16
BLK = 2000


def _mesh():
  return plsc.VectorSubcoreMesh(
      core_axis_name="c", subcore_axis_name="s",
      num_cores=NC, num_subcores=NS)


_SC_PARAMS = pltpu.CompilerParams(use_tc_tiling_on_sc=False)
_SC_PARAMS_NLP = pltpu.CompilerParams(
    use_tc_tiling_on_sc=False, needs_layout_passes=False)


def _deg_body(ept, dstf_hbm, out_hbm, idxf_v, hist_v, sum_v, rep_v, parts_sh):
  c = lax.axis_index("c")
  s = lax.axis_index("s")
  wid = c * NS + s
  pltpu.sync_copy(dstf_hbm.at[wid], idxf_v)

  @pl.loop(0, N_PAD // 16)
  def _(i):
    hist_v[pl.ds(i * 16, 16)] = jnp.zeros((16,), jnp.float32)

  ones16 = jnp.ones((16,), jnp.float32)

  @pl.loop(0, ept // 16)
  def _(i):
    ix = idxf_v[pl.ds(i * 16, 16)]
    plsc.addupdate_scatter(hist_v, [ix], ones16)

  pltpu.sync_copy(hist_v, parts_sh.at[s])
  plsc.subcore_barrier()
  for k in range(NS):
    pltpu.sync_copy(parts_sh.at[k, pl.ds(s * RPT, RPT)], sum_v.at[k])

  @pl.loop(0, RPT // 16)
  def _(i):
    acc = sum_v[0, pl.ds(i * 16, 16)]
    for k in range(1, NS):
      acc = acc + sum_v[k, pl.ds(i * 16, 16)]
    rows = i * 16 + lax.iota(jnp.int32, 16)
    for k in range(DEG_W):
      plsc.store_scatter(
          rep_v, [rows, jnp.full((16,), k, jnp.int32)], acc)

  pltpu.sync_copy(rep_v, out_hbm.at[c, pl.ds(s * RPT, RPT)])


def _agg_body(n_chunks, eidx_hbm, tbl_hbm, zeros_hbm, out_hbm,
              sidx_v, didx_v, rows_v, tbl_sh, acc_sh, gsems, ssems, psems):
  c = lax.axis_index("c")
  s = lax.axis_index("s")
  wid = c * NS + s
  pltpu.async_copy(zeros_hbm.at[pl.ds(s * RPT, RPT)],
                   acc_sh.at[pl.ds(s * RPT, RPT)], psems.at[0])
  pltpu.async_copy(tbl_hbm.at[pl.ds(s * RPT, RPT)],
                   tbl_sh.at[pl.ds(s * RPT, RPT)], psems.at[1])
  pltpu.sync_copy(eidx_hbm.at[0, wid], sidx_v)
  pltpu.sync_copy(eidx_hbm.at[1, wid], didx_v)
  pltpu.make_async_copy(zeros_hbm.at[pl.ds(s * RPT, RPT)],
                        acc_sh.at[pl.ds(s * RPT, RPT)], psems.at[0]).wait()
  pltpu.make_async_copy(tbl_hbm.at[pl.ds(s * RPT, RPT)],
                        tbl_sh.at[pl.ds(s * RPT, RPT)], psems.at[1]).wait()
  plsc.subcore_barrier()

  def _gather(j, slot):
    return pltpu.make_async_copy(
        tbl_sh.at[sidx_v.at[j]], rows_v.at[slot], gsems.at[slot])

  def _scat_start(j, slot):
    pltpu.async_copy(rows_v.at[slot], acc_sh.at[didx_v.at[j]],
                     ssems.at[slot], add=True)

  def _scat_wait(j, slot):
    pltpu.make_async_copy(rows_v.at[slot], acc_sh.at[didx_v.at[j]],
                          ssems.at[slot]).wait()

  _gather(0, 0).start()
  _gather(1, 1).start()

  @pl.loop(0, n_chunks)
  def _(j):
    slot = lax.rem(j, 3)
    _gather(j, slot).wait()
    _scat_start(j, slot)
    nj = j + 2

    @pl.when(nj < n_chunks)
    def _():
      ns = lax.rem(nj, 3)

      @pl.when(j >= 1)
      def _():
        _scat_wait(j - 1, ns)

      _gather(nj, ns).start()

  for t in range(3):
    j = n_chunks - 3 + t
    _scat_wait(j, j % 3)
  plsc.subcore_barrier()
  pltpu.sync_copy(acc_sh.at[pl.ds(s * RPT, RPT)],
                  out_hbm.at[c, pl.ds(s * RPT, RPT)])


def _make_deg_kernel(ept):
  return pl.kernel(
      functools.partial(_deg_body, ept),
      out_type=jax.ShapeDtypeStruct((NC, N_PAD, DEG_W), jnp.float32),
      mesh=_mesh(),
      scratch_types=[
          pltpu.VMEM((ept,), jnp.int32),
          pltpu.VMEM((N_PAD,), jnp.float32),
          pltpu.VMEM((NS, RPT), jnp.float32),
          pltpu.VMEM((RPT, DEG_W), jnp.float32),
          pltpu.VMEM_SHARED((NS, N_PAD), jnp.float32),
      ],
      compiler_params=_SC_PARAMS_NLP,
  )


def _make_agg_kernel(n_chunks):
  return pl.kernel(
      functools.partial(_agg_body, n_chunks),
      out_type=jax.ShapeDtypeStruct((NC, N_PAD, F_H), jnp.float32),
      mesh=_mesh(),
      scratch_types=[
          pltpu.VMEM((n_chunks, CHUNK), jnp.int32),
          pltpu.VMEM((n_chunks, CHUNK), jnp.int32),
          pltpu.VMEM((3, CHUNK, F_H), jnp.float32),
          pltpu.VMEM_SHARED((N_PAD, F_H), jnp.float32),
          pltpu.VMEM_SHARED((N_PAD, F_H), jnp.float32),
          pltpu.SemaphoreType.DMA((3,)),
          pltpu.SemaphoreType.DMA((3,)),
          pltpu.SemaphoreType.DMA((2,)),
      ],
      compiler_params=_SC_PARAMS,
  )


def _dinv_from(degp):
  deg = degp[0, :, 0:1] + degp[1, :, 0:1] + 1.0
  return lax.rsqrt(deg)


def _mm_body(x_ref, w_ref, out_ref):
  out_ref[...] = jnp.dot(
      x_ref[...], w_ref[...], preferred_element_type=jnp.float32)


def _scale_body(h_ref, degp_ref, out_ref):
  out_ref[...] = h_ref[...] * _dinv_from(degp_ref[...])


def _tc2_body(q_ref, hp_ref, degp_ref, b_ref, w_ref, out_ref):
  dinv = _dinv_from(degp_ref[...])
  ssum = q_ref[0] + q_ref[1] + hp_ref[...]
  h = jnp.maximum(ssum * dinv + b_ref[...], 0.0)
  out_ref[...] = jnp.dot(
      h, w_ref[...], preferred_element_type=jnp.float32) * dinv


def _tc3_body(q_ref, hp_ref, degp_ref, b_ref, wfc_ref, bfc_ref, out_ref):
  dinv = _dinv_from(degp_ref[...])
  ssum = q_ref[0] + q_ref[1] + hp_ref[...]
  h = jnp.maximum(ssum * dinv + b_ref[...], 0.0)
  z = jnp.dot(h, wfc_ref[...], preferred_element_type=jnp.float32)
  z = z + bfc_ref[...]
  m = jnp.max(z, axis=1, keepdims=True)
  lse = jnp.log(jnp.sum(jnp.exp(z - m), axis=1, keepdims=True)) + m
  out_ref[...] = z - lse


def _row_spec(w):
  return pl.BlockSpec((BLK, w), lambda i: (i, 0))


def _full_spec(shape):
  nd = len(shape)
  return pl.BlockSpec(shape, lambda i: (0,) * nd)


_DEGP_SPEC = pl.BlockSpec((NC, BLK, DEG_W), lambda i: (0, i, 0))
_Q_SPEC = pl.BlockSpec((NC, BLK, F_H), lambda i: (0, i, 0))
_GRID = (N // BLK,)


def _tc_mm(x, W1):
  return pl.pallas_call(
      _mm_body,
      grid=_GRID,
      in_specs=[_row_spec(F_IN), _full_spec((F_IN, F_H))],
      out_specs=_row_spec(F_H),
      out_shape=jax.ShapeDtypeStruct((N, F_H), jnp.float32),
  )(x, W1)


def _tc_scale(h, degp):
  return pl.pallas_call(
      _scale_body,
      grid=_GRID,
      in_specs=[_row_spec(F_H), _DEGP_SPEC],
      out_specs=_row_spec(F_H),
      out_shape=jax.ShapeDtypeStruct((N_PAD, F_H), jnp.float32),
  )(h, degp)


def _tc2(q, hp, degp, b1, W2):
  return pl.pallas_call(
      _tc2_body,
      grid=_GRID,
      in_specs=[_Q_SPEC, _row_spec(F_H), _DEGP_SPEC,
                _full_spec((1, F_H)), _full_spec((F_H, F_H))],
      out_specs=_row_spec(F_H),
      out_shape=jax.ShapeDtypeStruct((N_PAD, F_H), jnp.float32),
  )(q, hp, degp, b1, W2)


def _tc3(q, hp, degp, b2, Wfc, bfc):
  return pl.pallas_call(
      _tc3_body,
      grid=_GRID,
      in_specs=[_Q_SPEC, _row_spec(F_H), _DEGP_SPEC,
                _full_spec((1, F_H)), _full_spec((F_H, F_OUT)),
                _full_spec((1, F_OUT))],
      out_specs=_row_spec(F_OUT),
      out_shape=jax.ShapeDtypeStruct((N, F_OUT), jnp.float32),
  )(q, hp, degp, b2, Wfc, bfc)


def kernel(x, edge_index, W1, b1, W2, b2, Wfc, bfc):
  e = edge_index.shape[1]
  ei = edge_index.astype(jnp.int32)

  per_round = NW * CHUNK
  n_chunks = -(-e // per_round)
  e_pad = n_chunks * per_round
  if e_pad != e:
    fill = jnp.full((2, e_pad - e), N, jnp.int32)
    ei = jnp.concatenate([ei, fill], axis=1)
  eidx = ei.reshape(2, NW, n_chunks, CHUNK)
  ept = n_chunks * CHUNK
  dstf = ei[1].reshape(NW, ept)
  if ept % 16:
    ep16 = -(-ept // 16) * 16
    dstf = jnp.concatenate(
        [dstf, jnp.full((NW, ep16 - ept), N, jnp.int32)], axis=1)
    ept = ep16

  zeros_h = jnp.zeros((N_PAD, F_H), jnp.float32)

  deg_kernel = _make_deg_kernel(ept)
  agg_kernel = _make_agg_kernel(n_chunks)

  h1 = _tc_mm(x, W1)
  degp = deg_kernel(dstf)
  h1p = _tc_scale(h1, degp)
  q1 = agg_kernel(eidx, h1p, zeros_h)
  h2p = _tc2(q1, h1p, degp, b1[None, :], W2)
  q2 = agg_kernel(eidx, h2p, zeros_h)
  return _tc3(q2, h2p, degp, b2[None, :], Wfc, bfc[None, :])


# speedup vs baseline: 1.0437x; 1.0437x over previous
import functools

import jax
import jax.numpy as jnp
from jax import lax
from jax.experimental import pallas as pl
from jax.experimental.pallas import tpu as pltpu
from jax.experimental.pallas import tpu_sc as plsc

N = 10000
N_PAD = 10240
F_IN = 128
F_H = 64
F_OUT = 32
NC = 2
NS = 16
NW = NC * NS
CHUNK = 125
RPT = N_PAD // NS
DEG_W = 16
BLK = 2000


def _mesh():
  return plsc.VectorSubcoreMesh(
      core_axis_name="c", subcore_axis_name="s",
      num_cores=NC, num_subcores=NS)


_SC_PARAMS = pltpu.CompilerParams(use_tc_tiling_on_sc=False)
_SC_PARAMS_NLP = pltpu.CompilerParams(
    use_tc_tiling_on_sc=False, needs_layout_passes=False)


def _deg_body(n_chunks, eidx_hbm, out_hbm, idx_v, hist_v, sum_v, rep_v,
              parts_sh):
  c = lax.axis_index("c")
  s = lax.axis_index("s")
  wid = c * NS + s
  pltpu.sync_copy(eidx_hbm.at[1, wid], idx_v)

  @pl.loop(0, N_PAD // 16)
  def _(i):
    hist_v[pl.ds(i * 16, 16)] = jnp.zeros((16,), jnp.float32)

  ones16 = jnp.ones((16,), jnp.float32)
  n_full = CHUNK // 16
  tail = CHUNK % 16

  @pl.loop(0, n_chunks)
  def _(j):
    for g in range(n_full):
      ix = idx_v[j, pl.ds(g * 16, 16)]
      plsc.addupdate_scatter(hist_v, [ix], ones16)
    if tail:
      ix = idx_v[j, pl.ds(CHUNK - 16, 16)]
      mask = lax.iota(jnp.int32, 16) >= (16 - tail)
      plsc.addupdate_scatter(hist_v, [ix], ones16, mask=mask)

  pltpu.sync_copy(hist_v, parts_sh.at[s])
  plsc.subcore_barrier()
  for k in range(NS):
    pltpu.sync_copy(parts_sh.at[k, pl.ds(s * RPT, RPT)], sum_v.at[k])

  @pl.loop(0, RPT // 16)
  def _(i):
    acc = sum_v[0, pl.ds(i * 16, 16)]
    for k in range(1, NS):
      acc = acc + sum_v[k, pl.ds(i * 16, 16)]
    rows = i * 16 + lax.iota(jnp.int32, 16)
    for k in range(DEG_W):
      plsc.store_scatter(
          rep_v, [rows, jnp.full((16,), k, jnp.int32)], acc)

  pltpu.sync_copy(rep_v, out_hbm.at[c, pl.ds(s * RPT, RPT)])


def _agg_body(n_chunks, eidx_hbm, tbl_hbm, zeros_hbm, out_hbm,
              sidx_v, didx_v, rows_v, tbl_sh, acc_sh, gsems, ssems, psems):
  c = lax.axis_index("c")
  s = lax.axis_index("s")
  wid = c * NS + s
  pltpu.async_copy(zeros_hbm.at[pl.ds(s * RPT, RPT)],
                   acc_sh.at[pl.ds(s * RPT, RPT)], psems.at[0])
  pltpu.async_copy(tbl_hbm.at[pl.ds(s * RPT, RPT)],
                   tbl_sh.at[pl.ds(s * RPT, RPT)], psems.at[1])
  pltpu.sync_copy(eidx_hbm.at[0, wid], sidx_v)
  pltpu.sync_copy(eidx_hbm.at[1, wid], didx_v)
  pltpu.make_async_copy(zeros_hbm.at[pl.ds(s * RPT, RPT)],
                        acc_sh.at[pl.ds(s * RPT, RPT)], psems.at[0]).wait()
  pltpu.make_async_copy(tbl_hbm.at[pl.ds(s * RPT, RPT)],
                        tbl_sh.at[pl.ds(s * RPT, RPT)], psems.at[1]).wait()
  plsc.subcore_barrier()

  def _gather(j, slot):
    return pltpu.make_async_copy(
        tbl_sh.at[sidx_v.at[j]], rows_v.at[slot], gsems.at[slot])

  def _scat_start(j, slot):
    pltpu.async_copy(rows_v.at[slot], acc_sh.at[didx_v.at[j]],
                     ssems.at[slot], add=True)

  def _scat_wait(j, slot):
    pltpu.make_async_copy(rows_v.at[slot], acc_sh.at[didx_v.at[j]],
                          ssems.at[slot]).wait()

  _gather(0, 0).start()
  _gather(1, 1).start()

  @pl.loop(0, n_chunks)
  def _(j):
    slot = lax.rem(j, 3)
    _gather(j, slot).wait()
    _scat_start(j, slot)
    nj = j + 2

    @pl.when(nj < n_chunks)
    def _():
      ns = lax.rem(nj, 3)

      @pl.when(j >= 1)
      def _():
        _scat_wait(j - 1, ns)

      _gather(nj, ns).start()

  for t in range(3):
    j = n_chunks - 3 + t
    _scat_wait(j, j % 3)
  plsc.subcore_barrier()
  pltpu.sync_copy(acc_sh.at[pl.ds(s * RPT, RPT)],
                  out_hbm.at[c, pl.ds(s * RPT, RPT)])


def _make_deg_kernel(n_chunks):
  return pl.kernel(
      functools.partial(_deg_body, n_chunks),
      out_type=jax.ShapeDtypeStruct((NC, N_PAD, DEG_W), jnp.float32),
      mesh=_mesh(),
      scratch_types=[
          pltpu.VMEM((n_chunks, CHUNK), jnp.int32),
          pltpu.VMEM((N_PAD,), jnp.float32),
          pltpu.VMEM((NS, RPT), jnp.float32),
          pltpu.VMEM((RPT, DEG_W), jnp.float32),
          pltpu.VMEM_SHARED((NS, N_PAD), jnp.float32),
      ],
      compiler_params=_SC_PARAMS_NLP,
  )


def _make_agg_kernel(n_chunks):
  return pl.kernel(
      functools.partial(_agg_body, n_chunks),
      out_type=jax.ShapeDtypeStruct((NC, N_PAD, F_H), jnp.float32),
      mesh=_mesh(),
      scratch_types=[
          pltpu.VMEM((n_chunks, CHUNK), jnp.int32),
          pltpu.VMEM((n_chunks, CHUNK), jnp.int32),
          pltpu.VMEM((3, CHUNK, F_H), jnp.float32),
          pltpu.VMEM_SHARED((N_PAD, F_H), jnp.float32),
          pltpu.VMEM_SHARED((N_PAD, F_H), jnp.float32),
          pltpu.SemaphoreType.DMA((3,)),
          pltpu.SemaphoreType.DMA((3,)),
          pltpu.SemaphoreType.DMA((2,)),
      ],
      compiler_params=_SC_PARAMS,
  )


def _dinv_from(degp):
  deg = degp[0, :, 0:1] + degp[1, :, 0:1] + 1.0
  return lax.rsqrt(deg)


def _mm_body(x_ref, w_ref, out_ref):
  out_ref[...] = jnp.dot(
      x_ref[...], w_ref[...], preferred_element_type=jnp.float32)


def _scale_body(h_ref, degp_ref, out_ref):
  out_ref[...] = h_ref[...] * _dinv_from(degp_ref[...])


def _tc2_body(q_ref, hp_ref, degp_ref, b_ref, w_ref, out_ref):
  dinv = _dinv_from(degp_ref[...])
  ssum = q_ref[0] + q_ref[1] + hp_ref[...]
  h = jnp.maximum(ssum * dinv + b_ref[...], 0.0)
  out_ref[...] = jnp.dot(
      h, w_ref[...], preferred_element_type=jnp.float32) * dinv


def _tc3_body(q_ref, hp_ref, degp_ref, b_ref, wfc_ref, bfc_ref, out_ref):
  dinv = _dinv_from(degp_ref[...])
  ssum = q_ref[0] + q_ref[1] + hp_ref[...]
  h = jnp.maximum(ssum * dinv + b_ref[...], 0.0)
  z = jnp.dot(h, wfc_ref[...], preferred_element_type=jnp.float32)
  z = z + bfc_ref[...]
  m = jnp.max(z, axis=1, keepdims=True)
  lse = jnp.log(jnp.sum(jnp.exp(z - m), axis=1, keepdims=True)) + m
  out_ref[...] = z - lse


def _row_spec(w):
  return pl.BlockSpec((BLK, w), lambda i: (i, 0))


def _full_spec(shape):
  nd = len(shape)
  return pl.BlockSpec(shape, lambda i: (0,) * nd)


_DEGP_SPEC = pl.BlockSpec((NC, BLK, DEG_W), lambda i: (0, i, 0))
_Q_SPEC = pl.BlockSpec((NC, BLK, F_H), lambda i: (0, i, 0))
_GRID = (N // BLK,)


def _tc_mm(x, W1):
  return pl.pallas_call(
      _mm_body,
      grid=_GRID,
      in_specs=[_row_spec(F_IN), _full_spec((F_IN, F_H))],
      out_specs=_row_spec(F_H),
      out_shape=jax.ShapeDtypeStruct((N, F_H), jnp.float32),
  )(x, W1)


def _tc_scale(h, degp):
  return pl.pallas_call(
      _scale_body,
      grid=_GRID,
      in_specs=[_row_spec(F_H), _DEGP_SPEC],
      out_specs=_row_spec(F_H),
      out_shape=jax.ShapeDtypeStruct((N_PAD, F_H), jnp.float32),
  )(h, degp)


def _tc2(q, hp, degp, b1, W2):
  return pl.pallas_call(
      _tc2_body,
      grid=_GRID,
      in_specs=[_Q_SPEC, _row_spec(F_H), _DEGP_SPEC,
                _full_spec((1, F_H)), _full_spec((F_H, F_H))],
      out_specs=_row_spec(F_H),
      out_shape=jax.ShapeDtypeStruct((N_PAD, F_H), jnp.float32),
  )(q, hp, degp, b1, W2)


def _tc3(q, hp, degp, b2, Wfc, bfc):
  return pl.pallas_call(
      _tc3_body,
      grid=_GRID,
      in_specs=[_Q_SPEC, _row_spec(F_H), _DEGP_SPEC,
                _full_spec((1, F_H)), _full_spec((F_H, F_OUT)),
                _full_spec((1, F_OUT))],
      out_specs=_row_spec(F_OUT),
      out_shape=jax.ShapeDtypeStruct((N, F_OUT), jnp.float32),
  )(q, hp, degp, b2, Wfc, bfc)


def kernel(x, edge_index, W1, b1, W2, b2, Wfc, bfc):
  e = edge_index.shape[1]
  ei = edge_index.astype(jnp.int32)

  per_round = NW * CHUNK
  n_chunks = -(-e // per_round)
  e_pad = n_chunks * per_round
  if e_pad != e:
    fill = jnp.full((2, e_pad - e), N, jnp.int32)
    ei = jnp.concatenate([ei, fill], axis=1)
  eidx = ei.reshape(2, NW, n_chunks, CHUNK)

  zeros_h = jnp.zeros((N_PAD, F_H), jnp.float32)

  deg_kernel = _make_deg_kernel(n_chunks)
  agg_kernel = _make_agg_kernel(n_chunks)

  h1 = _tc_mm(x, W1)
  degp = deg_kernel(eidx)
  h1p = _tc_scale(h1, degp)
  q1 = agg_kernel(eidx, h1p, zeros_h)
  h2p = _tc2(q1, h1p, degp, b1[None, :], W2)
  q2 = agg_kernel(eidx, h2p, zeros_h)
  return _tc3(q2, h2p, degp, b2[None, :], Wfc, bfc[None, :])
